# baseline (speedup 1.0000x reference)
import functools
import math

import jax
import jax.numpy as jnp
import numpy as np
from jax import lax
from jax.experimental import pallas as pl
from jax.experimental.pallas import tpu as pltpu

DIM = 1024
BASE = 10000.0
ROWS_PER_BLOCK = 1024

_POLY = (
    6.206831691012579,
    -38.512967049333284,
    55.251554097285855,
)


def _pe_add_block(x_ref, tid_ref, o_ref):
    t = tid_ref[...].reshape(ROWS_PER_BLOCK, 1)
    dd = lax.broadcasted_iota(jnp.int32, (1, DIM), 1)
    even = dd & 1
    freqs = jnp.exp((dd - even).astype(jnp.float32) * (-math.log(BASE) / DIM)) * (
        1.0 / (2.0 * math.pi)
    )
    ph2 = even.astype(jnp.float32) * 0.25
    w = t * freqs + ph2
    d = w - jnp.round(w)
    u = d * d
    p = jnp.float32(_POLY[2])
    for c in _POLY[1::-1]:
        p = p * u + jnp.float32(c)
    o_ref[...] = x_ref[...] + p * d


@jax.jit
def kernel(x, time_ids):
    b, s, dim = x.shape
    n = b * s
    xf = x.reshape(n, dim)
    tf = time_ids.reshape(n).astype(jnp.float32)
    grid = n // ROWS_PER_BLOCK
    out = pl.pallas_call(
        _pe_add_block,
        grid=(grid,),
        in_specs=[
            pl.BlockSpec((ROWS_PER_BLOCK, dim), lambda i: (i, 0)),
            pl.BlockSpec((ROWS_PER_BLOCK,), lambda i: (i,)),
        ],
        out_specs=pl.BlockSpec((ROWS_PER_BLOCK, dim), lambda i: (i, 0)),
        out_shape=jax.ShapeDtypeStruct((n, dim), x.dtype),
        compiler_params=pltpu.CompilerParams(
            dimension_semantics=("arbitrary",),
        ),
    )(xf, tf)
    return out.reshape(b, s, dim)


# speedup vs baseline: 1.0306x; 1.0306x over previous
import math

import jax
import jax.numpy as jnp
from jax import lax
from jax.experimental import pallas as pl
from jax.experimental.pallas import tpu as pltpu

DIM = 1024
BASE = 10000.0
ROWS_PER_BLOCK = 2048

_POLY = (
    6.206831691012579,
    -38.512967049333284,
    55.251554097285855,
)


def _pe_add_block(x_ref, tid_ref, o_ref):
    t = tid_ref[...].reshape(ROWS_PER_BLOCK, 1)
    dd = lax.broadcasted_iota(jnp.int32, (1, DIM), 1)
    even = dd & 1
    freqs = jnp.exp((dd - even).astype(jnp.float32) * (-math.log(BASE) / DIM)) * (
        1.0 / (2.0 * math.pi)
    )
    ph2 = even.astype(jnp.float32) * 0.25
    w = t * freqs + ph2
    d = w - jnp.round(w)
    u = d * d
    p = jnp.float32(_POLY[2])
    for c in _POLY[1::-1]:
        p = p * u + jnp.float32(c)
    o_ref[...] = x_ref[...] + p * d


@jax.jit
def kernel(x, time_ids):
    b, s, dim = x.shape
    n = b * s
    xf = x.reshape(n, dim)
    tf = time_ids.reshape(n).astype(jnp.float32)
    grid = n // ROWS_PER_BLOCK
    out = pl.pallas_call(
        _pe_add_block,
        grid=(grid,),
        in_specs=[
            pl.BlockSpec((ROWS_PER_BLOCK, dim), lambda i: (i, 0)),
            pl.BlockSpec((ROWS_PER_BLOCK,), lambda i: (i,)),
        ],
        out_specs=pl.BlockSpec((ROWS_PER_BLOCK, dim), lambda i: (i, 0)),
        out_shape=jax.ShapeDtypeStruct((n, dim), x.dtype),
        compiler_params=pltpu.CompilerParams(
            dimension_semantics=("arbitrary",),
        ),
    )(xf, tf)
    return out.reshape(b, s, dim)
